# Initial kernel scaffold; baseline (speedup 1.0000x reference)
import jax
import jax.numpy as jnp
from jax.experimental import pallas as pl


def kernel(x, frames_label, num_classes, nmod):
    raise NotImplementedError("write your pallas kernel here")


# speedup vs baseline: 3.4004x; 3.4004x over previous
import functools

import jax
import jax.numpy as jnp
from jax.experimental import pallas as pl
from jax.experimental.pallas import tpu as pltpu

_KWS_LIST = (5, 12, 23, 37, 41, 58, 66, 74, 89, 97, 103, 118, 127, 134, 149, 155)
_NODE_NONE = 3002


def _loss_body(lb_ref, x_ref, posvec_ref, out_ref, acc_ref):
    i = pl.program_id(0)
    n = pl.num_programs(0)

    @pl.when(i == 0)
    def _init():
        for k in range(6):
            acc_ref[k] = 0.0

    xb = x_ref[...]
    lb = lb_ref[...]
    posvec = posvec_ref[...]

    r, c = xb.shape
    col = jax.lax.broadcasted_iota(jnp.int32, (r, c), 1)
    hit = col == lb
    m = jnp.max(xb, axis=1, keepdims=True)
    v = jnp.max(jnp.where(hit, xb, -jnp.inf), axis=1, keepdims=True)
    e = jnp.exp(xb - m)
    s = jnp.sum(e, axis=1, keepdims=True)
    ks = jnp.sum(e * posvec, axis=1, keepdims=True)
    lse = m + jnp.log(s)
    nll = lse - v

    is_kws = jnp.zeros_like(lb, dtype=jnp.bool_)
    for kc in _KWS_LIST:
        is_kws = is_kws | (lb == kc)
    nonneg = lb >= 0
    pos_row = is_kws & nonneg
    neg_row = (~is_kws) & nonneg & (lb != _NODE_NONE)

    zero = jnp.zeros_like(nll)
    one = jnp.ones_like(nll)
    acc_ref[0] += jnp.sum(jnp.where(pos_row, nll, zero))
    acc_ref[1] += jnp.sum(jnp.where(pos_row, one, zero))
    acc_ref[2] += jnp.sum(jnp.where(neg_row, nll, zero))
    acc_ref[3] += jnp.sum(jnp.where(neg_row, one, zero))

    pp = ks / s
    pn = (s - ks) / s
    lse2 = jnp.log(jnp.exp(pp) + jnp.exp(pn))
    bce = lse2 - jnp.where(is_kws, pp, pn)
    acc_ref[4] += jnp.sum(jnp.where(nonneg, bce, zero))
    acc_ref[5] += jnp.sum(jnp.where(nonneg, one, zero))

    @pl.when(i == n - 1)
    def _fin():
        sp, npos = acc_ref[0], acc_ref[1]
        sn, nneg = acc_ref[2], acc_ref[3]
        sb, nbce = acc_ref[4], acc_ref[5]
        loss = (sp / jnp.maximum(npos, 1.0)
                + sn / jnp.maximum(nneg, 1.0)
                + sb / jnp.maximum(nbce, 1.0))
        out_ref[...] = jnp.reshape(loss, (1, 1))


@functools.partial(jax.jit, static_argnums=())
def kernel(x, frames_label, num_classes, nmod):
    B, T, C = x.shape
    x2 = x.reshape(B * T, C)
    N = B * T
    nmod_static = frames_label.size // N
    lb = frames_label.reshape(N, nmod_static)[:, :1].astype(jnp.int32)

    posvec = jnp.zeros((1, C), dtype=x.dtype).at[0, jnp.array(_KWS_LIST)].set(1.0)

    R = 512
    grid = N // R
    out = pl.pallas_call(
        _loss_body,
        grid=(grid,),
        in_specs=[
            pl.BlockSpec((R, 1), lambda i: (i, 0)),
            pl.BlockSpec((R, C), lambda i: (i, 0)),
            pl.BlockSpec((1, C), lambda i: (0, 0)),
        ],
        out_specs=pl.BlockSpec((1, 1), lambda i: (0, 0)),
        out_shape=jax.ShapeDtypeStruct((1, 1), x.dtype),
        scratch_shapes=[pltpu.SMEM((6,), jnp.float32)],
        compiler_params=pltpu.CompilerParams(
            dimension_semantics=("arbitrary",),
        ),
    )(lb, x2, posvec)
    return out[0, 0]
